# Initial kernel scaffold; baseline (speedup 1.0000x reference)
import jax
import jax.numpy as jnp
from jax.experimental import pallas as pl


def kernel(cls_heads, reg_heads, batch_anchors):
    raise NotImplementedError("write your pallas kernel here")


# speedup vs baseline: 1.6210x; 1.6210x over previous
import jax
import jax.numpy as jnp
from jax import lax
from jax.experimental import pallas as pl

IMAGE_W = 1024
IMAGE_H = 1024
TOP_N = 1000
MIN_SCORE = 0.05
NMS_THR = 0.5
MAX_DET = 100

L, B, N, C = 5, 4, 20000, 80
NB = 10
BLK = N // NB
ROWS = (L * N + 127) // 128
PADDED = ROWS * 128
NEG_INF = float("-inf")
ONE_BITS = 0x3F800001


def _score_body(cls_ref, sc_ref, cl_ref):
    x = cls_ref[0, 0]
    m = jnp.max(x, axis=1, keepdims=True)
    li = lax.broadcasted_iota(jnp.int32, (BLK, C), 1)
    am = jnp.min(jnp.where(x == m, li, C + 1), axis=1)
    sc_ref[0, 0, 0, 0, :] = m[:, 0]
    cl_ref[0, 0, 0, 0, :] = am.astype(jnp.float32)


def _select_body(s_ref, o_ref):
    s = s_ref[...]
    bits = lax.bitcast_convert_type(s, jnp.int32)
    col = lax.broadcasted_iota(jnp.int32, (L * B, N), 1)

    def bs_step(_, lohi):
        lo, hi = lohi
        mid = lo + ((hi - lo) >> 1)
        c = jnp.sum((bits >= mid).astype(jnp.int32), axis=1, keepdims=True)
        ge = c >= TOP_N
        return jnp.where(ge, mid, lo), jnp.where(ge, hi, mid)

    lo0 = jnp.zeros((L * B, 1), jnp.int32)
    hi0 = jnp.full((L * B, 1), ONE_BITS, jnp.int32)
    vk, _ = lax.fori_loop(0, 31, bs_step, (lo0, hi0))

    c_gt = jnp.sum((bits > vk).astype(jnp.int32), axis=1, keepdims=True)
    need = TOP_N - c_gt
    eq = bits == vk

    def ix_step(_, lohi):
        lo, hi = lohi
        mid = lo + ((hi - lo) >> 1)
        g = jnp.sum((eq & (col < mid)).astype(jnp.int32), axis=1, keepdims=True)
        geq = g >= need
        return jnp.where(geq, lo, mid), jnp.where(geq, mid, hi)

    lo1 = jnp.zeros((L * B, 1), jnp.int32)
    hi1 = jnp.full((L * B, 1), N, jnp.int32)
    _, cut = lax.fori_loop(0, 15, ix_step, (lo1, hi1))

    sel = (bits > vk) | (eq & (col < cut))
    o_ref[...] = jnp.where(sel & (s > MIN_SCORE), s, NEG_INF)


def _nms_body(s_ref, cls_ref, rx_ref, ry_ref, rw_ref, rh_ref,
              ax1_ref, ay1_ref, ax2_ref, ay2_ref, out_ref):
    s = s_ref[0]
    clsf = cls_ref[0]
    ax1 = ax1_ref[0]
    ay1 = ay1_ref[0]
    ax2 = ax2_ref[0]
    ay2 = ay2_ref[0]

    awx = ax2 - ax1
    awy = ay2 - ay1
    acx = ax1 + 0.5 * awx
    acy = ay1 + 0.5 * awy
    whx = jnp.exp(rw_ref[0] * 0.2) * awx
    why = jnp.exp(rh_ref[0] * 0.2) * awy
    cx = rx_ref[0] * 0.1 * awx + acx
    cy = ry_ref[0] * 0.1 * awy + acy
    x1 = jnp.maximum((cx - 0.5 * whx).astype(jnp.int32), 0).astype(jnp.float32)
    y1 = jnp.maximum((cy - 0.5 * why).astype(jnp.int32), 0).astype(jnp.float32)
    x2 = jnp.minimum((cx + 0.5 * whx).astype(jnp.int32), IMAGE_W - 1).astype(jnp.float32)
    y2 = jnp.minimum((cy + 0.5 * why).astype(jnp.int32), IMAGE_H - 1).astype(jnp.float32)
    areas = (x2 - x1) * (y2 - y1)

    flat = (lax.broadcasted_iota(jnp.int32, (ROWS, 128), 0) * 128
            + lax.broadcasted_iota(jnp.int32, (ROWS, 128), 1))
    row8 = lax.broadcasted_iota(jnp.int32, (8, 128), 0)
    lane8 = lax.broadcasted_iota(jnp.int32, (8, 128), 1)

    out_ref[0] = jnp.zeros((8, 128), jnp.float32)

    def step(i, ms):
        m = jnp.max(ms)
        has = m > NEG_INF
        eqm = ms == m
        idx = jnp.min(jnp.where(eqm, flat, jnp.int32(1 << 30)))
        selm = flat == idx
        bx1 = jnp.sum(jnp.where(selm, x1, 0.0))
        by1 = jnp.sum(jnp.where(selm, y1, 0.0))
        bx2 = jnp.sum(jnp.where(selm, x2, 0.0))
        by2 = jnp.sum(jnp.where(selm, y2, 0.0))
        bc = jnp.sum(jnp.where(selm, clsf, 0.0))
        barea = (bx2 - bx1) * (by2 - by1)
        xx1 = jnp.maximum(x1, bx1)
        yy1 = jnp.maximum(y1, by1)
        xx2 = jnp.minimum(x2, bx2)
        yy2 = jnp.minimum(y2, by2)
        inter = jnp.maximum(xx2 - xx1, 0.0) * jnp.maximum(yy2 - yy1, 0.0)
        iou = inter / (areas + barea - inter)
        new_ms = jnp.where((iou > NMS_THR) | selm, NEG_INF, ms)
        ms = jnp.where(has, new_ms, ms)
        lm = lane8 == i
        vals = (jnp.where(has, m, -1.0), jnp.where(has, bc, -1.0),
                jnp.where(has, bx1, -1.0), jnp.where(has, by1, -1.0),
                jnp.where(has, bx2, -1.0), jnp.where(has, by2, -1.0))
        cur = out_ref[0]
        for r, v in enumerate(vals):
            cur = jnp.where(lm & (row8 == r), v, cur)
        out_ref[0] = cur
        return ms

    lax.fori_loop(0, MAX_DET, step, s)


def _interp():
    return jax.default_backend() == "cpu"


def kernel(cls_heads, reg_heads, batch_anchors):
    scores, classes = pl.pallas_call(
        _score_body,
        grid=(L, B, NB),
        in_specs=[pl.BlockSpec((1, 1, BLK, C), lambda l, b, n: (l, b, n, 0))],
        out_specs=[
            pl.BlockSpec((1, 1, 1, 1, BLK), lambda l, b, n: (b, l, n, 0, 0)),
            pl.BlockSpec((1, 1, 1, 1, BLK), lambda l, b, n: (b, l, n, 0, 0)),
        ],
        out_shape=[
            jax.ShapeDtypeStruct((B, L, NB, 1, BLK), jnp.float32),
            jax.ShapeDtypeStruct((B, L, NB, 1, BLK), jnp.float32),
        ],
        interpret=_interp(),
    )(cls_heads)

    lb_scores = scores.reshape(B * L, N)
    masked = pl.pallas_call(
        _select_body,
        out_shape=jax.ShapeDtypeStruct((B * L, N), jnp.float32),
        interpret=_interp(),
    )(lb_scores)

    def to_rows(x_bflat, fill):
        p = jnp.full((B, PADDED - L * N), fill, jnp.float32)
        return jnp.concatenate([x_bflat, p], axis=1).reshape(B, ROWS, 128)

    s_b = to_rows(masked.reshape(B, L * N), NEG_INF)
    cls_b = to_rows(classes.reshape(B, L * N), 0.0)

    def comp(x, j):
        return to_rows(x[..., j].transpose(1, 0, 2).reshape(B, L * N), 0.0)

    rx, ry, rw, rh = (comp(reg_heads, j) for j in range(4))
    ax1, ay1, ax2, ay2 = (comp(batch_anchors, j) for j in range(4))

    spec = pl.BlockSpec((1, ROWS, 128), lambda b: (b, 0, 0))
    out = pl.pallas_call(
        _nms_body,
        grid=(B,),
        in_specs=[spec] * 10,
        out_specs=pl.BlockSpec((1, 8, 128), lambda b: (b, 0, 0)),
        out_shape=jax.ShapeDtypeStruct((B, 8, 128), jnp.float32),
        interpret=_interp(),
    )(s_b, cls_b, rx, ry, rw, rh, ax1, ay1, ax2, ay2)

    batch_scores = out[:, 0, :MAX_DET]
    batch_classes = out[:, 1, :MAX_DET]
    batch_boxes = jnp.stack(
        [out[:, 2, :MAX_DET], out[:, 3, :MAX_DET],
         out[:, 4, :MAX_DET], out[:, 5, :MAX_DET]], axis=-1)
    return batch_scores, batch_classes, batch_boxes


# speedup vs baseline: 1.7172x; 1.0594x over previous
import jax
import jax.numpy as jnp
from jax import lax
from jax.experimental import pallas as pl

IMAGE_W = 1024
IMAGE_H = 1024
TOP_N = 1000
MIN_SCORE = 0.05
NMS_THR = 0.5
MAX_DET = 100

L, B, N, C = 5, 4, 20000, 80
NB = 10
BLK = N // NB
ROWS = (L * N + 127) // 128
PADDED = ROWS * 128
NEG_INF = float("-inf")
ONE_BITS = 0x3F800001


def _score_body(cls_ref, sc_ref, cl_ref):
    x = cls_ref[0, 0]
    m = jnp.max(x, axis=1, keepdims=True)
    li = lax.broadcasted_iota(jnp.int32, (BLK, C), 1)
    am = jnp.min(jnp.where(x == m, li, C + 1), axis=1)
    sc_ref[0, 0, 0, 0, :] = m[:, 0]
    cl_ref[0, 0, 0, 0, :] = am.astype(jnp.float32)


def _select_body(s_ref, o_ref):
    s = s_ref[...]
    bits = lax.bitcast_convert_type(s, jnp.int32)
    col = lax.broadcasted_iota(jnp.int32, (L * B, N), 1)

    def bs_step(_, lohi):
        lo, hi = lohi
        mid = lo + ((hi - lo) >> 1)
        c = jnp.sum((bits >= mid).astype(jnp.int32), axis=1, keepdims=True)
        ge = c >= TOP_N
        return jnp.where(ge, mid, lo), jnp.where(ge, hi, mid)

    lo0 = jnp.zeros((L * B, 1), jnp.int32)
    hi0 = jnp.full((L * B, 1), ONE_BITS, jnp.int32)
    vk, _ = lax.fori_loop(0, 31, bs_step, (lo0, hi0))

    c_gt = jnp.sum((bits > vk).astype(jnp.int32), axis=1, keepdims=True)
    need = TOP_N - c_gt
    eq = bits == vk

    def ix_step(_, lohi):
        lo, hi = lohi
        mid = lo + ((hi - lo) >> 1)
        g = jnp.sum((eq & (col < mid)).astype(jnp.int32), axis=1, keepdims=True)
        geq = g >= need
        return jnp.where(geq, lo, mid), jnp.where(geq, mid, hi)

    lo1 = jnp.zeros((L * B, 1), jnp.int32)
    hi1 = jnp.full((L * B, 1), N, jnp.int32)
    _, cut = lax.fori_loop(0, 15, ix_step, (lo1, hi1))

    sel = (bits > vk) | (eq & (col < cut))
    o_ref[...] = jnp.where(sel & (s > MIN_SCORE), s, NEG_INF)


def _nms_body(s_ref, cls_ref, rx_ref, ry_ref, rw_ref, rh_ref,
              ax1_ref, ay1_ref, ax2_ref, ay2_ref, out_ref):
    s = s_ref[0]
    clsf = cls_ref[0]
    ax1 = ax1_ref[0]
    ay1 = ay1_ref[0]
    ax2 = ax2_ref[0]
    ay2 = ay2_ref[0]

    awx = ax2 - ax1
    awy = ay2 - ay1
    acx = ax1 + 0.5 * awx
    acy = ay1 + 0.5 * awy
    whx = jnp.exp(rw_ref[0] * 0.2) * awx
    why = jnp.exp(rh_ref[0] * 0.2) * awy
    cx = rx_ref[0] * 0.1 * awx + acx
    cy = ry_ref[0] * 0.1 * awy + acy
    x1 = jnp.maximum((cx - 0.5 * whx).astype(jnp.int32), 0).astype(jnp.float32)
    y1 = jnp.maximum((cy - 0.5 * why).astype(jnp.int32), 0).astype(jnp.float32)
    x2 = jnp.minimum((cx + 0.5 * whx).astype(jnp.int32), IMAGE_W - 1).astype(jnp.float32)
    y2 = jnp.minimum((cy + 0.5 * why).astype(jnp.int32), IMAGE_H - 1).astype(jnp.float32)
    areas = (x2 - x1) * (y2 - y1)
    p1 = x1 + 4096.0 * y1
    p2 = x2 + 4096.0 * y2

    flat = (lax.broadcasted_iota(jnp.int32, (ROWS, 128), 0) * 128
            + lax.broadcasted_iota(jnp.int32, (ROWS, 128), 1))
    row8 = lax.broadcasted_iota(jnp.int32, (8, 128), 0)
    lane8 = lax.broadcasted_iota(jnp.int32, (8, 128), 1)

    out_ref[0] = jnp.zeros((8, 128), jnp.float32)

    def step(i, ms):
        m = jnp.max(ms)
        has = m > NEG_INF
        eqm = ms == m
        idx = jnp.min(jnp.where(eqm, flat, jnp.int32(1 << 30)))
        selm = flat == idx
        bp1 = jnp.sum(jnp.where(selm, p1, 0.0))
        bp2 = jnp.sum(jnp.where(selm, p2, 0.0))
        bc = jnp.sum(jnp.where(selm, clsf, 0.0))
        by1 = jnp.floor(bp1 * (1.0 / 4096.0))
        bx1 = bp1 - 4096.0 * by1
        by2 = jnp.floor(bp2 * (1.0 / 4096.0))
        bx2 = bp2 - 4096.0 * by2
        barea = (bx2 - bx1) * (by2 - by1)
        xx1 = jnp.maximum(x1, bx1)
        yy1 = jnp.maximum(y1, by1)
        xx2 = jnp.minimum(x2, bx2)
        yy2 = jnp.minimum(y2, by2)
        inter = jnp.maximum(xx2 - xx1, 0.0) * jnp.maximum(yy2 - yy1, 0.0)
        supp = (3.0 * inter > areas + barea) | selm
        ms = jnp.where(supp, NEG_INF, ms)
        lm = lane8 == i
        vals = (jnp.where(has, m, -1.0), jnp.where(has, bc, -1.0),
                jnp.where(has, bx1, -1.0), jnp.where(has, by1, -1.0),
                jnp.where(has, bx2, -1.0), jnp.where(has, by2, -1.0))
        cur = out_ref[0]
        for r, v in enumerate(vals):
            cur = jnp.where(lm & (row8 == r), v, cur)
        out_ref[0] = cur
        return ms

    lax.fori_loop(0, MAX_DET, step, s)


def _interp():
    return jax.default_backend() == "cpu"


def kernel(cls_heads, reg_heads, batch_anchors):
    scores, classes = pl.pallas_call(
        _score_body,
        grid=(L, B, NB),
        in_specs=[pl.BlockSpec((1, 1, BLK, C), lambda l, b, n: (l, b, n, 0))],
        out_specs=[
            pl.BlockSpec((1, 1, 1, 1, BLK), lambda l, b, n: (b, l, n, 0, 0)),
            pl.BlockSpec((1, 1, 1, 1, BLK), lambda l, b, n: (b, l, n, 0, 0)),
        ],
        out_shape=[
            jax.ShapeDtypeStruct((B, L, NB, 1, BLK), jnp.float32),
            jax.ShapeDtypeStruct((B, L, NB, 1, BLK), jnp.float32),
        ],
        interpret=_interp(),
    )(cls_heads)

    lb_scores = scores.reshape(B * L, N)
    masked = pl.pallas_call(
        _select_body,
        out_shape=jax.ShapeDtypeStruct((B * L, N), jnp.float32),
        interpret=_interp(),
    )(lb_scores)

    def to_rows(x_bflat, fill):
        p = jnp.full((B, PADDED - L * N), fill, jnp.float32)
        return jnp.concatenate([x_bflat, p], axis=1).reshape(B, ROWS, 128)

    s_b = to_rows(masked.reshape(B, L * N), NEG_INF)
    cls_b = to_rows(classes.reshape(B, L * N), 0.0)

    def comp(x, j):
        return to_rows(x[..., j].transpose(1, 0, 2).reshape(B, L * N), 0.0)

    rx, ry, rw, rh = (comp(reg_heads, j) for j in range(4))
    ax1, ay1, ax2, ay2 = (comp(batch_anchors, j) for j in range(4))

    spec = pl.BlockSpec((1, ROWS, 128), lambda b: (b, 0, 0))
    out = pl.pallas_call(
        _nms_body,
        grid=(B,),
        in_specs=[spec] * 10,
        out_specs=pl.BlockSpec((1, 8, 128), lambda b: (b, 0, 0)),
        out_shape=jax.ShapeDtypeStruct((B, 8, 128), jnp.float32),
        interpret=_interp(),
    )(s_b, cls_b, rx, ry, rw, rh, ax1, ay1, ax2, ay2)

    batch_scores = out[:, 0, :MAX_DET]
    batch_classes = out[:, 1, :MAX_DET]
    batch_boxes = jnp.stack(
        [out[:, 2, :MAX_DET], out[:, 3, :MAX_DET],
         out[:, 4, :MAX_DET], out[:, 5, :MAX_DET]], axis=-1)
    return batch_scores, batch_classes, batch_boxes


# speedup vs baseline: 1.7660x; 1.0284x over previous
import jax
import jax.numpy as jnp
from jax import lax
from jax.experimental import pallas as pl

IMAGE_W = 1024
IMAGE_H = 1024
TOP_N = 1000
MIN_SCORE = 0.05
NMS_THR = 0.5
MAX_DET = 100

L, B, N, C = 5, 4, 20000, 80
NB = 10
BLK = N // NB
ROWS = (L * N + 127) // 128
PADDED = ROWS * 128
NEG_INF = float("-inf")
ONE_BITS = 0x3F800001


def _score_body(cls_ref, sc_ref, cl_ref):
    x = cls_ref[0, 0]
    m = jnp.max(x, axis=1, keepdims=True)
    li = lax.broadcasted_iota(jnp.int32, (BLK, C), 1)
    am = jnp.min(jnp.where(x == m, li, C + 1), axis=1)
    sc_ref[0, 0, 0, 0, :] = m[:, 0]
    cl_ref[0, 0, 0, 0, :] = am.astype(jnp.float32)


def _select_body(s_ref, o_ref):
    s = s_ref[...]
    bits = lax.bitcast_convert_type(s, jnp.int32)
    col = lax.broadcasted_iota(jnp.int32, (L * B, N), 1)

    def bs_step(_, lohi):
        lo, hi = lohi
        mid = lo + ((hi - lo) >> 1)
        c = jnp.sum((bits >= mid).astype(jnp.int32), axis=1, keepdims=True)
        ge = c >= TOP_N
        return jnp.where(ge, mid, lo), jnp.where(ge, hi, mid)

    lo0 = jnp.zeros((L * B, 1), jnp.int32)
    hi0 = jnp.full((L * B, 1), ONE_BITS, jnp.int32)
    vk, _ = lax.fori_loop(0, 31, bs_step, (lo0, hi0))

    c_gt = jnp.sum((bits > vk).astype(jnp.int32), axis=1, keepdims=True)
    need = TOP_N - c_gt
    eq = bits == vk

    def ix_step(_, lohi):
        lo, hi = lohi
        mid = lo + ((hi - lo) >> 1)
        g = jnp.sum((eq & (col < mid)).astype(jnp.int32), axis=1, keepdims=True)
        geq = g >= need
        return jnp.where(geq, lo, mid), jnp.where(geq, mid, hi)

    lo1 = jnp.zeros((L * B, 1), jnp.int32)
    hi1 = jnp.full((L * B, 1), N, jnp.int32)
    _, cut = lax.fori_loop(0, 15, ix_step, (lo1, hi1))

    sel = (bits > vk) | (eq & (col < cut))
    o_ref[...] = jnp.where(sel & (s > MIN_SCORE), s, NEG_INF)


def _nms_body(s_ref, cls_ref, rx_ref, ry_ref, rw_ref, rh_ref,
              ax1_ref, ay1_ref, ax2_ref, ay2_ref, out_ref):
    s = s_ref[0]
    clsf = cls_ref[0]
    ax1 = ax1_ref[0]
    ay1 = ay1_ref[0]
    ax2 = ax2_ref[0]
    ay2 = ay2_ref[0]

    awx = ax2 - ax1
    awy = ay2 - ay1
    acx = ax1 + 0.5 * awx
    acy = ay1 + 0.5 * awy
    whx = jnp.exp(rw_ref[0] * 0.2) * awx
    why = jnp.exp(rh_ref[0] * 0.2) * awy
    cx = rx_ref[0] * 0.1 * awx + acx
    cy = ry_ref[0] * 0.1 * awy + acy
    x1 = jnp.maximum((cx - 0.5 * whx).astype(jnp.int32), 0).astype(jnp.float32)
    y1 = jnp.maximum((cy - 0.5 * why).astype(jnp.int32), 0).astype(jnp.float32)
    x2 = jnp.minimum((cx + 0.5 * whx).astype(jnp.int32), IMAGE_W - 1).astype(jnp.float32)
    y2 = jnp.minimum((cy + 0.5 * why).astype(jnp.int32), IMAGE_H - 1).astype(jnp.float32)
    areas = (x2 - x1) * (y2 - y1)
    p1 = x1 + 4096.0 * y1
    p2 = x2 + 4096.0 * y2

    flat = (lax.broadcasted_iota(jnp.int32, (ROWS, 128), 0) * 128
            + lax.broadcasted_iota(jnp.int32, (ROWS, 128), 1))
    row8 = lax.broadcasted_iota(jnp.int32, (8, 128), 0)
    lane8 = lax.broadcasted_iota(jnp.int32, (8, 128), 1)

    out_ref[0] = jnp.full((8, 128), -1.0, jnp.float32)

    def cond(carry):
        _, i_out, has = carry
        return (i_out < MAX_DET) & has

    def body(carry):
        ms, i_out, _ = carry
        m = jnp.max(ms)
        has = m > NEG_INF
        eqm = ms == m
        idx = jnp.min(jnp.where(eqm, flat, jnp.int32(1 << 30)))
        selm = flat == idx
        bp1 = jnp.sum(jnp.where(selm, p1, 0.0))
        bp2 = jnp.sum(jnp.where(selm, p2, 0.0))
        bc = jnp.sum(jnp.where(selm, clsf, 0.0))
        by1 = jnp.floor(bp1 * (1.0 / 4096.0))
        bx1 = bp1 - 4096.0 * by1
        by2 = jnp.floor(bp2 * (1.0 / 4096.0))
        bx2 = bp2 - 4096.0 * by2
        barea = (bx2 - bx1) * (by2 - by1)
        cur = out_ref[0]
        sx1 = cur[2:3, :]
        sy1 = cur[3:4, :]
        sx2 = cur[4:5, :]
        sy2 = cur[5:6, :]
        iw = jnp.minimum(sx2, bx2) - jnp.maximum(sx1, bx1)
        ih = jnp.minimum(sy2, by2) - jnp.maximum(sy1, by1)
        inter = jnp.maximum(iw, 0.0) * jnp.maximum(ih, 0.0)
        sarea = (sx2 - sx1) * (sy2 - sy1)
        is_supp = jnp.any(3.0 * inter > sarea + barea)
        ms = jnp.where(selm, NEG_INF, ms)
        take = has & ~is_supp
        lm = (lane8 == i_out) & take
        vals = (m, bc, bx1, by1, bx2, by2)
        for r, v in enumerate(vals):
            cur = jnp.where(lm & (row8 == r), v, cur)
        out_ref[0] = cur
        i_out = lax.select(take, i_out + 1, i_out)
        return ms, i_out, has

    lax.while_loop(cond, body, (s, jnp.int32(0), jnp.bool_(True)))


def _interp():
    return jax.default_backend() == "cpu"


def kernel(cls_heads, reg_heads, batch_anchors):
    scores, classes = pl.pallas_call(
        _score_body,
        grid=(L, B, NB),
        in_specs=[pl.BlockSpec((1, 1, BLK, C), lambda l, b, n: (l, b, n, 0))],
        out_specs=[
            pl.BlockSpec((1, 1, 1, 1, BLK), lambda l, b, n: (b, l, n, 0, 0)),
            pl.BlockSpec((1, 1, 1, 1, BLK), lambda l, b, n: (b, l, n, 0, 0)),
        ],
        out_shape=[
            jax.ShapeDtypeStruct((B, L, NB, 1, BLK), jnp.float32),
            jax.ShapeDtypeStruct((B, L, NB, 1, BLK), jnp.float32),
        ],
        interpret=_interp(),
    )(cls_heads)

    lb_scores = scores.reshape(B * L, N)
    masked = pl.pallas_call(
        _select_body,
        out_shape=jax.ShapeDtypeStruct((B * L, N), jnp.float32),
        interpret=_interp(),
    )(lb_scores)

    def to_rows(x_bflat, fill):
        p = jnp.full((B, PADDED - L * N), fill, jnp.float32)
        return jnp.concatenate([x_bflat, p], axis=1).reshape(B, ROWS, 128)

    s_b = to_rows(masked.reshape(B, L * N), NEG_INF)
    cls_b = to_rows(classes.reshape(B, L * N), 0.0)

    def comp(x, j):
        return to_rows(x[..., j].transpose(1, 0, 2).reshape(B, L * N), 0.0)

    rx, ry, rw, rh = (comp(reg_heads, j) for j in range(4))
    ax1, ay1, ax2, ay2 = (comp(batch_anchors, j) for j in range(4))

    spec = pl.BlockSpec((1, ROWS, 128), lambda b: (b, 0, 0))
    out = pl.pallas_call(
        _nms_body,
        grid=(B,),
        in_specs=[spec] * 10,
        out_specs=pl.BlockSpec((1, 8, 128), lambda b: (b, 0, 0)),
        out_shape=jax.ShapeDtypeStruct((B, 8, 128), jnp.float32),
        interpret=_interp(),
    )(s_b, cls_b, rx, ry, rw, rh, ax1, ay1, ax2, ay2)

    batch_scores = out[:, 0, :MAX_DET]
    batch_classes = out[:, 1, :MAX_DET]
    batch_boxes = jnp.stack(
        [out[:, 2, :MAX_DET], out[:, 3, :MAX_DET],
         out[:, 4, :MAX_DET], out[:, 5, :MAX_DET]], axis=-1)
    return batch_scores, batch_classes, batch_boxes


# speedup vs baseline: 1.7916x; 1.0145x over previous
import jax
import jax.numpy as jnp
from jax import lax
from jax.experimental import pallas as pl
from jax.experimental.pallas import tpu as pltpu

IMAGE_W = 1024
IMAGE_H = 1024
TOP_N = 1000
MIN_SCORE = 0.05
NMS_THR = 0.5
MAX_DET = 100

L, B, N, C = 5, 4, 20000, 80
NB = 10
BLK = N // NB
ROWS = (L * N + 127) // 128
PADDED = ROWS * 128
NEG_INF = float("-inf")
ONE_BITS = 0x3F800001


def _score_body(cls_ref, sc_ref, cl_ref):
    x = cls_ref[0, 0]
    m = jnp.max(x, axis=1, keepdims=True)
    li = lax.broadcasted_iota(jnp.int32, (BLK, C), 1)
    am = jnp.min(jnp.where(x == m, li, C + 1), axis=1, keepdims=True)
    sc_ref[0, 0] = m
    cl_ref[0, 0] = am.astype(jnp.float32)


def _select_body(s_ref, o_ref):
    s = s_ref[...]
    bits = lax.bitcast_convert_type(s, jnp.int32)
    col = lax.broadcasted_iota(jnp.int32, (L * B, N), 1)

    def bs_step(_, lohi):
        lo, hi = lohi
        mid = lo + ((hi - lo) >> 1)
        c = jnp.sum((bits >= mid).astype(jnp.int32), axis=1, keepdims=True)
        ge = c >= TOP_N
        return jnp.where(ge, mid, lo), jnp.where(ge, hi, mid)

    lo0 = jnp.zeros((L * B, 1), jnp.int32)
    hi0 = jnp.full((L * B, 1), ONE_BITS, jnp.int32)
    vk, _ = lax.fori_loop(0, 31, bs_step, (lo0, hi0))

    c_gt = jnp.sum((bits > vk).astype(jnp.int32), axis=1, keepdims=True)
    need = TOP_N - c_gt
    eq = bits == vk

    def ix_step(_, lohi):
        lo, hi = lohi
        mid = lo + ((hi - lo) >> 1)
        g = jnp.sum((eq & (col < mid)).astype(jnp.int32), axis=1, keepdims=True)
        geq = g >= need
        return jnp.where(geq, lo, mid), jnp.where(geq, mid, hi)

    lo1 = jnp.zeros((L * B, 1), jnp.int32)
    hi1 = jnp.full((L * B, 1), N, jnp.int32)
    _, cut = lax.fori_loop(0, 15, ix_step, (lo1, hi1))

    sel = (bits > vk) | (eq & (col < cut))
    o_ref[...] = jnp.where(sel & (s > MIN_SCORE), s, NEG_INF)


def _nms_body(s_ref, cls_ref, rx_ref, ry_ref, rw_ref, rh_ref,
              ax1_ref, ay1_ref, ax2_ref, ay2_ref, out_ref):
    s = s_ref[0]
    clsf = cls_ref[0]
    ax1 = ax1_ref[0]
    ay1 = ay1_ref[0]
    ax2 = ax2_ref[0]
    ay2 = ay2_ref[0]

    awx = ax2 - ax1
    awy = ay2 - ay1
    acx = ax1 + 0.5 * awx
    acy = ay1 + 0.5 * awy
    whx = jnp.exp(rw_ref[0] * 0.2) * awx
    why = jnp.exp(rh_ref[0] * 0.2) * awy
    cx = rx_ref[0] * 0.1 * awx + acx
    cy = ry_ref[0] * 0.1 * awy + acy
    x1 = jnp.maximum((cx - 0.5 * whx).astype(jnp.int32), 0).astype(jnp.float32)
    y1 = jnp.maximum((cy - 0.5 * why).astype(jnp.int32), 0).astype(jnp.float32)
    x2 = jnp.minimum((cx + 0.5 * whx).astype(jnp.int32), IMAGE_W - 1).astype(jnp.float32)
    y2 = jnp.minimum((cy + 0.5 * why).astype(jnp.int32), IMAGE_H - 1).astype(jnp.float32)
    areas = (x2 - x1) * (y2 - y1)
    p1 = x1 + 4096.0 * y1
    p2 = x2 + 4096.0 * y2

    flat = (lax.broadcasted_iota(jnp.int32, (ROWS, 128), 0) * 128
            + lax.broadcasted_iota(jnp.int32, (ROWS, 128), 1))
    row8 = lax.broadcasted_iota(jnp.int32, (8, 128), 0)
    lane8 = lax.broadcasted_iota(jnp.int32, (8, 128), 1)

    out_ref[0] = jnp.full((8, 128), -1.0, jnp.float32)

    def cond(carry):
        _, i_out, has = carry
        return (i_out < MAX_DET) & has

    def body(carry):
        ms, i_out, _ = carry
        m = jnp.max(ms)
        has = m > NEG_INF
        eqm = ms == m
        idx = jnp.min(jnp.where(eqm, flat, jnp.int32(1 << 30)))
        selm = flat == idx
        bp1 = jnp.sum(jnp.where(selm, p1, 0.0))
        bp2 = jnp.sum(jnp.where(selm, p2, 0.0))
        bc = jnp.sum(jnp.where(selm, clsf, 0.0))
        by1 = jnp.floor(bp1 * (1.0 / 4096.0))
        bx1 = bp1 - 4096.0 * by1
        by2 = jnp.floor(bp2 * (1.0 / 4096.0))
        bx2 = bp2 - 4096.0 * by2
        barea = (bx2 - bx1) * (by2 - by1)
        cur = out_ref[0]
        sx1 = cur[2:3, :]
        sy1 = cur[3:4, :]
        sx2 = cur[4:5, :]
        sy2 = cur[5:6, :]
        iw = jnp.minimum(sx2, bx2) - jnp.maximum(sx1, bx1)
        ih = jnp.minimum(sy2, by2) - jnp.maximum(sy1, by1)
        inter = jnp.maximum(iw, 0.0) * jnp.maximum(ih, 0.0)
        sarea = (sx2 - sx1) * (sy2 - sy1)
        is_supp = jnp.any(3.0 * inter > sarea + barea)
        ms = jnp.where(selm, NEG_INF, ms)
        take = has & ~is_supp
        lm = (lane8 == i_out) & take
        vals = (m, bc, bx1, by1, bx2, by2)
        for r, v in enumerate(vals):
            cur = jnp.where(lm & (row8 == r), v, cur)
        out_ref[0] = cur
        i_out = lax.select(take, i_out + 1, i_out)
        return ms, i_out, has

    lax.while_loop(cond, body, (s, jnp.int32(0), jnp.bool_(True)))


def kernel(cls_heads, reg_heads, batch_anchors):
    scores, classes = pl.pallas_call(
        _score_body,
        grid=(L, B, NB),
        in_specs=[pl.BlockSpec((1, 1, BLK, C), lambda l, b, n: (l, b, n, 0))],
        out_specs=[
            pl.BlockSpec((1, 1, BLK, 1), lambda l, b, n: (b, l, n, 0)),
            pl.BlockSpec((1, 1, BLK, 1), lambda l, b, n: (b, l, n, 0)),
        ],
        out_shape=[
            jax.ShapeDtypeStruct((B, L, N, 1), jnp.float32),
            jax.ShapeDtypeStruct((B, L, N, 1), jnp.float32),
        ],
        compiler_params=pltpu.CompilerParams(
            dimension_semantics=("parallel", "parallel", "parallel")),
    )(cls_heads)

    lb_scores = scores.reshape(B * L, N)
    masked = pl.pallas_call(
        _select_body,
        out_shape=jax.ShapeDtypeStruct((B * L, N), jnp.float32),
    )(lb_scores)

    def to_rows(x_bflat, fill):
        p = jnp.full((B, PADDED - L * N), fill, jnp.float32)
        return jnp.concatenate([x_bflat, p], axis=1).reshape(B, ROWS, 128)

    s_b = to_rows(masked.reshape(B, L * N), NEG_INF)
    cls_b = to_rows(classes.reshape(B, L * N), 0.0)

    def comp(x, j):
        return to_rows(x[..., j].transpose(1, 0, 2).reshape(B, L * N), 0.0)

    rx, ry, rw, rh = (comp(reg_heads, j) for j in range(4))
    ax1, ay1, ax2, ay2 = (comp(batch_anchors, j) for j in range(4))

    spec = pl.BlockSpec((1, ROWS, 128), lambda b: (b, 0, 0))
    out = pl.pallas_call(
        _nms_body,
        grid=(B,),
        in_specs=[spec] * 10,
        out_specs=pl.BlockSpec((1, 8, 128), lambda b: (b, 0, 0)),
        out_shape=jax.ShapeDtypeStruct((B, 8, 128), jnp.float32),
        compiler_params=pltpu.CompilerParams(
            dimension_semantics=("parallel",)),
    )(s_b, cls_b, rx, ry, rw, rh, ax1, ay1, ax2, ay2)

    batch_scores = out[:, 0, :MAX_DET]
    batch_classes = out[:, 1, :MAX_DET]
    batch_boxes = jnp.stack(
        [out[:, 2, :MAX_DET], out[:, 3, :MAX_DET],
         out[:, 4, :MAX_DET], out[:, 5, :MAX_DET]], axis=-1)
    return batch_scores, batch_classes, batch_boxes


# speedup vs baseline: 1.8835x; 1.0513x over previous
import jax
import jax.numpy as jnp
from jax import lax
from jax.experimental import pallas as pl
from jax.experimental.pallas import tpu as pltpu

IMAGE_W = 1024
IMAGE_H = 1024
TOP_N = 1000
MIN_SCORE = 0.05
NMS_THR = 0.5
MAX_DET = 100

L, B, N, C = 5, 4, 20000, 80
NB = 10
BLK = N // NB
ROWS = (L * N + 127) // 128
PADDED = ROWS * 128
NEG_INF = float("-inf")
ONE_BITS = 0x3F800001


def _score_body(cls_ref, sc_ref, cl_ref):
    x = cls_ref[0, 0]
    m = jnp.max(x, axis=1, keepdims=True)
    li = lax.broadcasted_iota(jnp.int32, (BLK, C), 1)
    am = jnp.min(jnp.where(x == m, li, C + 1), axis=1, keepdims=True)
    sc_ref[0, 0] = m
    cl_ref[0, 0] = am.astype(jnp.float32)


def _select_body(s_ref, o_ref):
    s = s_ref[...]
    bits = lax.bitcast_convert_type(s, jnp.int32)
    col = lax.broadcasted_iota(jnp.int32, (L * B, N), 1)

    def bs_step(_, lohi):
        lo, hi = lohi
        mid = lo + ((hi - lo) >> 1)
        c = jnp.sum((bits >= mid).astype(jnp.int32), axis=1, keepdims=True)
        ge = c >= TOP_N
        return jnp.where(ge, mid, lo), jnp.where(ge, hi, mid)

    lo0 = jnp.zeros((L * B, 1), jnp.int32)
    hi0 = jnp.full((L * B, 1), ONE_BITS, jnp.int32)
    vk, _ = lax.fori_loop(0, 31, bs_step, (lo0, hi0))

    c_gt = jnp.sum((bits > vk).astype(jnp.int32), axis=1, keepdims=True)
    need = TOP_N - c_gt
    eq = bits == vk

    def ix_step(_, lohi):
        lo, hi = lohi
        mid = lo + ((hi - lo) >> 1)
        g = jnp.sum((eq & (col < mid)).astype(jnp.int32), axis=1, keepdims=True)
        geq = g >= need
        return jnp.where(geq, lo, mid), jnp.where(geq, mid, hi)

    lo1 = jnp.zeros((L * B, 1), jnp.int32)
    hi1 = jnp.full((L * B, 1), N, jnp.int32)
    _, cut = lax.fori_loop(0, 15, ix_step, (lo1, hi1))

    sel = (bits > vk) | (eq & (col < cut))
    o_ref[...] = jnp.where(sel & (s > MIN_SCORE), s, NEG_INF)


def _nms_body(s_ref, cls_ref, rx_ref, ry_ref, rw_ref, rh_ref,
              ax1_ref, ay1_ref, ax2_ref, ay2_ref, out_ref):
    imgs = []
    for b in range(B):
        imgs.append(_nms_prep(
            s_ref[b], cls_ref[b], rx_ref[b], ry_ref[b], rw_ref[b],
            rh_ref[b], ax1_ref[b], ay1_ref[b], ax2_ref[b], ay2_ref[b]))

    flat = (lax.broadcasted_iota(jnp.int32, (ROWS, 128), 0) * 128
            + lax.broadcasted_iota(jnp.int32, (ROWS, 128), 1))
    row8 = lax.broadcasted_iota(jnp.int32, (8, 128), 0)
    lane8 = lax.broadcasted_iota(jnp.int32, (8, 128), 1)

    for b in range(B):
        out_ref[b] = jnp.full((8, 128), -1.0, jnp.float32)

    def cond(carry):
        ms_l, io_l, has_l = carry
        go = (io_l[0] < MAX_DET) & has_l[0]
        for b in range(1, B):
            go = go | ((io_l[b] < MAX_DET) & has_l[b])
        return go

    def body(carry):
        ms_l, io_l, has_l = carry
        new_ms, new_io, new_has = [], [], []
        for b in range(B):
            ms, i_out = ms_l[b], io_l[b]
            _, p1, p2, clsf = imgs[b]
            m = jnp.max(ms)
            has = m > NEG_INF
            eqm = ms == m
            idx = jnp.min(jnp.where(eqm, flat, jnp.int32(1 << 30)))
            selm = flat == idx
            bp1 = jnp.sum(jnp.where(selm, p1, 0.0))
            bp2 = jnp.sum(jnp.where(selm, p2, 0.0))
            bc = jnp.sum(jnp.where(selm, clsf, 0.0))
            by1 = jnp.floor(bp1 * (1.0 / 4096.0))
            bx1 = bp1 - 4096.0 * by1
            by2 = jnp.floor(bp2 * (1.0 / 4096.0))
            bx2 = bp2 - 4096.0 * by2
            barea = (bx2 - bx1) * (by2 - by1)
            cur = out_ref[b]
            sx1 = cur[2:3, :]
            sy1 = cur[3:4, :]
            sx2 = cur[4:5, :]
            sy2 = cur[5:6, :]
            iw = jnp.minimum(sx2, bx2) - jnp.maximum(sx1, bx1)
            ih = jnp.minimum(sy2, by2) - jnp.maximum(sy1, by1)
            inter = jnp.maximum(iw, 0.0) * jnp.maximum(ih, 0.0)
            sarea = (sx2 - sx1) * (sy2 - sy1)
            is_supp = jnp.any(3.0 * inter > sarea + barea)
            ms = jnp.where(selm, NEG_INF, ms)
            take = has & ~is_supp & (i_out < MAX_DET)
            lm = (lane8 == i_out) & take
            vals = (m, bc, bx1, by1, bx2, by2)
            for r, v in enumerate(vals):
                cur = jnp.where(lm & (row8 == r), v, cur)
            out_ref[b] = cur
            new_ms.append(ms)
            new_io.append(lax.select(take, i_out + 1, i_out))
            new_has.append(has)
        return new_ms, new_io, new_has

    lax.while_loop(cond, body, (
        [imgs[b][0] for b in range(B)],
        [jnp.int32(0)] * B, [jnp.bool_(True)] * B))


def _nms_prep(s, clsf, rx, ry, rw, rh, ax1, ay1, ax2, ay2):
    awx = ax2 - ax1
    awy = ay2 - ay1
    acx = ax1 + 0.5 * awx
    acy = ay1 + 0.5 * awy
    whx = jnp.exp(rw * 0.2) * awx
    why = jnp.exp(rh * 0.2) * awy
    cx = rx * 0.1 * awx + acx
    cy = ry * 0.1 * awy + acy
    x1 = jnp.maximum((cx - 0.5 * whx).astype(jnp.int32), 0).astype(jnp.float32)
    y1 = jnp.maximum((cy - 0.5 * why).astype(jnp.int32), 0).astype(jnp.float32)
    x2 = jnp.minimum((cx + 0.5 * whx).astype(jnp.int32), IMAGE_W - 1).astype(jnp.float32)
    y2 = jnp.minimum((cy + 0.5 * why).astype(jnp.int32), IMAGE_H - 1).astype(jnp.float32)
    p1 = x1 + 4096.0 * y1
    p2 = x2 + 4096.0 * y2
    return s, p1, p2, clsf


def kernel(cls_heads, reg_heads, batch_anchors):
    scores, classes = pl.pallas_call(
        _score_body,
        grid=(L, B, NB),
        in_specs=[pl.BlockSpec((1, 1, BLK, C), lambda l, b, n: (l, b, n, 0))],
        out_specs=[
            pl.BlockSpec((1, 1, BLK, 1), lambda l, b, n: (b, l, n, 0)),
            pl.BlockSpec((1, 1, BLK, 1), lambda l, b, n: (b, l, n, 0)),
        ],
        out_shape=[
            jax.ShapeDtypeStruct((B, L, N, 1), jnp.float32),
            jax.ShapeDtypeStruct((B, L, N, 1), jnp.float32),
        ],
        compiler_params=pltpu.CompilerParams(
            dimension_semantics=("parallel", "parallel", "parallel")),
    )(cls_heads)

    lb_scores = scores.reshape(B * L, N)
    masked = pl.pallas_call(
        _select_body,
        out_shape=jax.ShapeDtypeStruct((B * L, N), jnp.float32),
    )(lb_scores)

    def to_rows(x_bflat, fill):
        p = jnp.full((B, PADDED - L * N), fill, jnp.float32)
        return jnp.concatenate([x_bflat, p], axis=1).reshape(B, ROWS, 128)

    s_b = to_rows(masked.reshape(B, L * N), NEG_INF)
    cls_b = to_rows(classes.reshape(B, L * N), 0.0)

    def comp(x, j):
        return to_rows(x[..., j].transpose(1, 0, 2).reshape(B, L * N), 0.0)

    rx, ry, rw, rh = (comp(reg_heads, j) for j in range(4))
    ax1, ay1, ax2, ay2 = (comp(batch_anchors, j) for j in range(4))

    out = pl.pallas_call(
        _nms_body,
        out_shape=jax.ShapeDtypeStruct((B, 8, 128), jnp.float32),
    )(s_b, cls_b, rx, ry, rw, rh, ax1, ay1, ax2, ay2)

    batch_scores = out[:, 0, :MAX_DET]
    batch_classes = out[:, 1, :MAX_DET]
    batch_boxes = jnp.stack(
        [out[:, 2, :MAX_DET], out[:, 3, :MAX_DET],
         out[:, 4, :MAX_DET], out[:, 5, :MAX_DET]], axis=-1)
    return batch_scores, batch_classes, batch_boxes
